# baseline (speedup 1.0000x reference)
import functools

import jax
import jax.numpy as jnp
from jax import lax
from jax.experimental import pallas as pl
from jax.experimental.pallas import tpu as pltpu
from jax.experimental.pallas import tpu_sc as plsc

N_NODES = 50000
N_EDGES = 800000
NODE_DIM = 64
EDGE_DIM = 32

N_HALF = N_EDGES // 2
NW = 32
EDGES_PER_W = N_HALF // NW
OP_ROWS = 125
K_OPS = 4
CHUNK = OP_ROWS * K_OPS
N_CHUNKS = EDGES_PER_W // CHUNK

_mesh = plsc.VectorSubcoreMesh(core_axis_name="c", subcore_axis_name="s")


@functools.partial(
    pl.kernel,
    mesh=_mesh,
    compiler_params=pltpu.CompilerParams(use_tc_tiling_on_sc=False),
    out_type=jax.ShapeDtypeStruct((N_HALF, 2 * EDGE_DIM), jnp.float32),
    scratch_types=[
        pltpu.VMEM((K_OPS, OP_ROWS), jnp.int32),
        pltpu.VMEM((K_OPS, OP_ROWS), jnp.int32),
        pltpu.VMEM((CHUNK, EDGE_DIM), jnp.float32),
        pltpu.VMEM((CHUNK, EDGE_DIM), jnp.float32),
        pltpu.SemaphoreType.DMA,
    ],
)
def _gather_edges(ptab_hbm, qtab_hbm, ports_hbm, protos_hbm, out_hbm,
                  idxp_v, idxq_v, bufp_v, bufq_v, sem):
    wid = lax.axis_index("s") * 2 + lax.axis_index("c")
    base = wid * EDGES_PER_W

    def body(i, carry):
        off = pl.multiple_of(base + i * CHUNK, CHUNK)
        idx_off = pl.multiple_of(
            wid * (EDGES_PER_W // OP_ROWS) + i * K_OPS, K_OPS)
        pltpu.sync_copy(ports_hbm.at[pl.ds(idx_off, K_OPS)], idxp_v)
        pltpu.sync_copy(protos_hbm.at[pl.ds(idx_off, K_OPS)], idxq_v)
        handles = []
        for j in range(K_OPS):
            sl = pl.ds(j * OP_ROWS, OP_ROWS)
            handles.append(pltpu.async_copy(
                ptab_hbm.at[idxp_v.at[j]], bufp_v.at[sl], sem))
            handles.append(pltpu.async_copy(
                qtab_hbm.at[idxq_v.at[j]], bufq_v.at[sl], sem))
        for h in handles:
            h.wait()
        rows = out_hbm.at[pl.ds(off, CHUNK)]
        pltpu.sync_copy(bufp_v, rows.at[:, pl.ds(0, EDGE_DIM)])
        pltpu.sync_copy(bufq_v, rows.at[:, pl.ds(EDGE_DIM, EDGE_DIM)])
        return carry

    lax.fori_loop(0, N_CHUNKS, body, 0)


def _mm_body(x_ref, w_ref, b_ref, o_ref):
    o_ref[...] = (
        lax.dot_general(
            x_ref[...], w_ref[...],
            (((1,), (1,)), ((), ())),
            preferred_element_type=jnp.float32,
        )
        + b_ref[...]
    )


_MM_BLOCK = 2000


def _node_proj(ip_bits, W_ip, b_ip):
    return pl.pallas_call(
        _mm_body,
        grid=(N_NODES // _MM_BLOCK,),
        in_specs=[
            pl.BlockSpec((_MM_BLOCK, 32), lambda i: (i, 0)),
            pl.BlockSpec((NODE_DIM, 32), lambda i: (0, 0)),
            pl.BlockSpec((1, NODE_DIM), lambda i: (0, 0)),
        ],
        out_specs=pl.BlockSpec((_MM_BLOCK, NODE_DIM), lambda i: (i, 0)),
        out_shape=jax.ShapeDtypeStruct((N_NODES, NODE_DIM), jnp.float32),
    )(ip_bits, W_ip, b_ip.reshape(1, NODE_DIM))


def kernel(ip_bits, ports, protos, W_ip, b_ip, port_table, proto_table):
    ports2 = ports.astype(jnp.int32).reshape(N_EDGES // OP_ROWS, OP_ROWS)
    protos2 = protos.astype(jnp.int32).reshape(N_EDGES // OP_ROWS, OP_ROWS)
    hrows = N_HALF // OP_ROWS
    e1 = _gather_edges(port_table, proto_table,
                       ports2[:hrows], protos2[:hrows])
    e2 = _gather_edges(port_table, proto_table,
                       ports2[hrows:], protos2[hrows:])
    edge_attr = jnp.concatenate([e1, e2], axis=0)
    x_embedded = _node_proj(ip_bits, W_ip, b_ip)
    return (x_embedded, edge_attr)


# speedup vs baseline: 1.2180x; 1.2180x over previous
import functools

import jax
import jax.numpy as jnp
from jax import lax
from jax.experimental import pallas as pl
from jax.experimental.pallas import tpu as pltpu
from jax.experimental.pallas import tpu_sc as plsc

N_NODES = 50000
N_EDGES = 800000
NODE_DIM = 64
EDGE_DIM = 32

NW = 32
EDGES_PER_W = N_EDGES // NW
OP_ROWS = 125
K_OPS = 8
CHUNK = OP_ROWS * K_OPS
N_CHUNKS = EDGES_PER_W // CHUNK

_mesh = plsc.VectorSubcoreMesh(core_axis_name="c", subcore_axis_name="s")


@functools.partial(
    pl.kernel,
    mesh=_mesh,
    compiler_params=pltpu.CompilerParams(use_tc_tiling_on_sc=False),
    out_type=jax.ShapeDtypeStruct((N_EDGES, 2 * EDGE_DIM), jnp.float32),
    scratch_types=[
        pltpu.VMEM((K_OPS, OP_ROWS), jnp.int32),
        pltpu.VMEM((K_OPS, OP_ROWS), jnp.int32),
        pltpu.VMEM((CHUNK, EDGE_DIM), jnp.float32),
        pltpu.VMEM((CHUNK, EDGE_DIM), jnp.float32),
        pltpu.SemaphoreType.DMA,
    ],
)
def _gather_edges(ptab_hbm, qtab_hbm, ports_hbm, protos_hbm, out_hbm,
                  idxp_v, idxq_v, bufp_v, bufq_v, sem):
    wid = lax.axis_index("s") * 2 + lax.axis_index("c")
    base = wid * EDGES_PER_W

    def body(i, carry):
        off = pl.multiple_of(base + i * CHUNK, CHUNK)
        idx_off = pl.multiple_of(
            wid * (EDGES_PER_W // OP_ROWS) + i * K_OPS, K_OPS)
        pltpu.sync_copy(ports_hbm.at[pl.ds(idx_off, K_OPS)], idxp_v)
        pltpu.sync_copy(protos_hbm.at[pl.ds(idx_off, K_OPS)], idxq_v)
        handles = []
        for j in range(K_OPS):
            sl = pl.ds(j * OP_ROWS, OP_ROWS)
            handles.append(pltpu.async_copy(
                ptab_hbm.at[idxp_v.at[j]], bufp_v.at[sl], sem))
            handles.append(pltpu.async_copy(
                qtab_hbm.at[idxq_v.at[j]], bufq_v.at[sl], sem))
        for h in handles:
            h.wait()
        rows = out_hbm.at[pl.ds(off, CHUNK)]
        pltpu.sync_copy(bufp_v, rows.at[:, pl.ds(0, EDGE_DIM)])
        pltpu.sync_copy(bufq_v, rows.at[:, pl.ds(EDGE_DIM, EDGE_DIM)])
        return carry

    lax.fori_loop(0, N_CHUNKS, body, 0)


def _mm_body(x_ref, w_ref, b_ref, o_ref):
    o_ref[...] = (
        lax.dot_general(
            x_ref[...], w_ref[...],
            (((1,), (1,)), ((), ())),
            preferred_element_type=jnp.float32,
        )
        + b_ref[...]
    )


_MM_BLOCK = 2000


def _node_proj(ip_bits, W_ip, b_ip):
    return pl.pallas_call(
        _mm_body,
        grid=(N_NODES // _MM_BLOCK,),
        in_specs=[
            pl.BlockSpec((_MM_BLOCK, 32), lambda i: (i, 0)),
            pl.BlockSpec((NODE_DIM, 32), lambda i: (0, 0)),
            pl.BlockSpec((1, NODE_DIM), lambda i: (0, 0)),
        ],
        out_specs=pl.BlockSpec((_MM_BLOCK, NODE_DIM), lambda i: (i, 0)),
        out_shape=jax.ShapeDtypeStruct((N_NODES, NODE_DIM), jnp.float32),
    )(ip_bits, W_ip, b_ip.reshape(1, NODE_DIM))


def kernel(ip_bits, ports, protos, W_ip, b_ip, port_table, proto_table):
    ports2 = ports.astype(jnp.int32).reshape(N_EDGES // OP_ROWS, OP_ROWS)
    protos2 = protos.astype(jnp.int32).reshape(N_EDGES // OP_ROWS, OP_ROWS)
    edge_attr = _gather_edges(port_table, proto_table, ports2, protos2)
    x_embedded = _node_proj(ip_bits, W_ip, b_ip)
    return (x_embedded, edge_attr)
